# Initial kernel scaffold; baseline (speedup 1.0000x reference)
import jax
import jax.numpy as jnp
from jax.experimental import pallas as pl


def kernel(x):
    raise NotImplementedError("write your pallas kernel here")


# speedup vs baseline: 2.1632x; 2.1632x over previous
import jax
import jax.numpy as jnp
from jax.experimental import pallas as pl

NCLS = 1000
BLK = 1024


def _onehot_block(x_ref, o_ref):
    idx = x_ref[...]
    col = jax.lax.broadcasted_iota(jnp.int32, (BLK, NCLS), 1)
    o_ref[...] = (col == idx).astype(jnp.float32)


def kernel(x):
    xf = x.reshape(-1, 1).astype(jnp.int32)
    n = xf.shape[0]
    out = pl.pallas_call(
        _onehot_block,
        grid=(n // BLK,),
        in_specs=[pl.BlockSpec((BLK, 1), lambda i: (i, 0))],
        out_specs=pl.BlockSpec((BLK, NCLS), lambda i: (i, 0)),
        out_shape=jax.ShapeDtypeStruct((n, NCLS), jnp.float32),
    )(xf)
    return out.reshape(tuple(x.shape) + (NCLS,))
